# baseline (speedup 1.0000x reference)
import functools

import jax
import jax.numpy as jnp
from jax import lax
from jax.experimental import pallas as pl
from jax.experimental.pallas import tpu as pltpu
from jax.experimental.pallas import tpu_sc as plsc

N_USER = 1000000
N_ITEM = 1000000
K = 16
BATCH = 16384

NC = 2
NS = 16
L = 16
NW = NC * NS
BPW = BATCH // NW
CHUNK = 128
NCHUNK = BPW // CHUNK
NBLK = BPW // L
SUB = CHUNK // L

PITCH = 1000064
BMAIN = (N_USER // CHUNK) * CHUNK
BTAIL = N_USER - BMAIN

TBLK = 16384
TGRID = (PITCH + TBLK - 1) // TBLK

_mesh = plsc.VectorSubcoreMesh(
    core_axis_name="c", subcore_axis_name="s", num_cores=NC, num_subcores=NS
)


def _dt_body(ut_ref, vt_ref, u_ref, v_ref):
    u_ref[...] = ut_ref[...]
    v_ref[...] = vt_ref[...]


_detile = pl.pallas_call(
    _dt_body,
    grid=(TGRID,),
    in_specs=[
        pl.BlockSpec((K, TBLK), lambda j: (0, j)),
        pl.BlockSpec((K, TBLK), lambda j: (0, j)),
    ],
    out_specs=[
        pl.BlockSpec((K, TBLK), lambda j: (0, j)),
        pl.BlockSpec((K, TBLK), lambda j: (0, j)),
    ],
    out_shape=[
        jax.ShapeDtypeStruct((K, PITCH), jnp.float32),
        jax.ShapeDtypeStruct((K, PITCH), jnp.float32),
    ],
)


@functools.partial(
    pl.kernel,
    out_type=jax.ShapeDtypeStruct((BATCH,), jnp.float32),
    mesh=_mesh,
    compiler_params=pltpu.CompilerParams(
        needs_layout_passes=False, use_tc_tiling_on_sc=False
    ),
    scratch_types=dict(
        uidx_v=pltpu.VMEM((NCHUNK, CHUNK), jnp.int32),
        iidx_v=pltpu.VMEM((NCHUNK, CHUNK), jnp.int32),
        uidx_k=pltpu.VMEM((K, NCHUNK, CHUNK), jnp.int32),
        iidx_k=pltpu.VMEM((K, NCHUNK, CHUNK), jnp.int32),
        uidx_c=pltpu.VMEM((NCHUNK, CHUNK), jnp.int32),
        iidx_c=pltpu.VMEM((NCHUNK, CHUNK), jnp.int32),
        u_cols=pltpu.VMEM((K, BPW), jnp.float32),
        v_cols=pltpu.VMEM((K, BPW), jnp.float32),
        bu_rows=pltpu.VMEM((BPW,), jnp.float32),
        bi_rows=pltpu.VMEM((BPW,), jnp.float32),
        bu_tail=pltpu.VMEM((BTAIL,), jnp.float32),
        bi_tail=pltpu.VMEM((BTAIL,), jnp.float32),
        bias_v=pltpu.VMEM((L,), jnp.float32),
        out_v=pltpu.VMEM((BPW,), jnp.float32),
        sem=pltpu.SemaphoreType.DMA,
    ),
)
def _mf_sc(
    uidx_hbm,
    iidx_hbm,
    u_flat,
    v_flat,
    bu_main_hbm,
    bi_main_hbm,
    bu_tail_hbm,
    bi_tail_hbm,
    bias16,
    out_hbm,
    *,
    uidx_v,
    iidx_v,
    uidx_k,
    iidx_k,
    uidx_c,
    iidx_c,
    u_cols,
    v_cols,
    bu_rows,
    bi_rows,
    bu_tail,
    bi_tail,
    bias_v,
    out_v,
    sem,
):
    wid = lax.axis_index("s") * NC + lax.axis_index("c")

    pltpu.sync_copy(uidx_hbm.at[wid], uidx_v)
    pltpu.sync_copy(iidx_hbm.at[wid], iidx_v)
    pltpu.sync_copy(bu_tail_hbm, bu_tail)
    pltpu.sync_copy(bi_tail_hbm, bi_tail)
    pltpu.sync_copy(bias16, bias_v)

    bmax = jnp.full((L,), BMAIN - 1, jnp.int32)
    for c in range(NCHUNK):
        for b in range(SUB):
            s = pl.ds(b * L, L)
            base = uidx_v[c, s]
            uidx_c[c, s] = jnp.minimum(base, bmax)
            for k in range(K):
                uidx_k[k, c, s] = base + (k * PITCH)
            base = iidx_v[c, s]
            iidx_c[c, s] = jnp.minimum(base, bmax)
            for k in range(K):
                iidx_k[k, c, s] = base + (k * PITCH)

    descs = []
    for c in range(NCHUNK):
        rows = pl.ds(c * CHUNK, CHUNK)
        descs.append(pltpu.async_copy(bu_main_hbm.at[uidx_c.at[c]], bu_rows.at[rows], sem))
        descs.append(pltpu.async_copy(bi_main_hbm.at[iidx_c.at[c]], bi_rows.at[rows], sem))
        for k in range(K):
            descs.append(
                pltpu.async_copy(u_flat.at[uidx_k.at[k, c]], u_cols.at[k, rows], sem)
            )
            descs.append(
                pltpu.async_copy(v_flat.at[iidx_k.at[k, c]], v_cols.at[k, rows], sem)
            )
    for d in descs:
        d.wait()

    bias_vec = bias_v[...]
    bcut = jnp.full((L,), BMAIN, jnp.int32)
    tmax = jnp.full((L,), BTAIL - 1, jnp.int32)
    zero = jnp.zeros((L,), jnp.int32)

    def blk(i, carry):
        b = pl.ds(i * L, L)
        c = i // SUB
        s = pl.ds((i % SUB) * L, L)
        uix = uidx_v[c, s]
        iix = iidx_v[c, s]
        bu = bu_rows[b]
        bi = bi_rows[b]
        ut = plsc.load_gather(bu_tail, [jnp.clip(uix - bcut, zero, tmax)])
        it = plsc.load_gather(bi_tail, [jnp.clip(iix - bcut, zero, tmax)])
        bu = jnp.where(uix >= bcut, ut, bu)
        bi = jnp.where(iix >= bcut, it, bi)
        acc = bias_vec + bu + bi
        for k in range(K):
            acc = acc + u_cols[k, b] * v_cols[k, b]
        out_v[b] = acc
        return carry

    lax.fori_loop(0, NBLK, blk, 0)

    pltpu.sync_copy(out_v, out_hbm.at[pl.ds(wid * BPW, BPW)])


def kernel(train_x, user_emb, item_emb, bias_user, bias_item, bias):
    uidx = train_x[:, 0].reshape(NW, NCHUNK, CHUNK).astype(jnp.int32)
    iidx = train_x[:, 1].reshape(NW, NCHUNK, CHUNK).astype(jnp.int32)
    bias16 = jnp.broadcast_to(bias.astype(jnp.float32), (L,))
    u_dt, v_dt = _detile(user_emb.T, item_emb.T)
    u_flat = u_dt.reshape(K * PITCH)
    v_flat = v_dt.reshape(K * PITCH)
    bu_main = bias_user[:BMAIN].reshape(BMAIN)
    bi_main = bias_item[:BMAIN].reshape(BMAIN)
    bu_tail = bias_user[BMAIN:].reshape(BTAIL)
    bi_tail = bias_item[BMAIN:].reshape(BTAIL)
    return _mf_sc(
        uidx, iidx, u_flat, v_flat, bu_main, bi_main, bu_tail, bi_tail, bias16
    )


# speedup vs baseline: 11.4031x; 11.4031x over previous
import functools

import jax
import jax.numpy as jnp
from jax import lax
from jax.experimental import pallas as pl
from jax.experimental.pallas import tpu as pltpu
from jax.experimental.pallas import tpu_sc as plsc

N_USER = 1000000
N_ITEM = 1000000
K = 16
BATCH = 16384

NC = 2
NS = 16
L = 16
NW = NC * NS
BPW = BATCH // NW
CHUNK = 128
NCHUNK = BPW // CHUNK
NBLK = BPW // L
SUB = CHUNK // L

PITCH = 1000064
BMAIN = (N_USER // CHUNK) * CHUNK
BTAIL = N_USER - BMAIN

NTC = 7813
CB = 601
TBLK = CB * 128
TGRID = NTC // CB
HALF = NTC * 8 * 128

_mesh = plsc.VectorSubcoreMesh(
    core_axis_name="c", subcore_axis_name="s", num_cores=NC, num_subcores=NS
)


def _dt_body(ut_ref, vt_ref, u_ref, v_ref):
    def tiles(x_ref, o_ref):
        x = x_ref[...].reshape(8, CB, 128)
        y = jnp.transpose(x, (1, 0, 2))
        o_ref[...] = y.reshape(1, CB * 8, 128)

    tiles(ut_ref, u_ref)
    tiles(vt_ref, v_ref)


_detile = pl.pallas_call(
    _dt_body,
    grid=(2, TGRID),
    in_specs=[
        pl.BlockSpec((8, TBLK), lambda kb, j: (kb, j)),
        pl.BlockSpec((8, TBLK), lambda kb, j: (kb, j)),
    ],
    out_specs=[
        pl.BlockSpec((1, CB * 8, 128), lambda kb, j: (kb, j, 0)),
        pl.BlockSpec((1, CB * 8, 128), lambda kb, j: (kb, j, 0)),
    ],
    out_shape=[
        jax.ShapeDtypeStruct((2, NTC * 8, 128), jnp.float32),
        jax.ShapeDtypeStruct((2, NTC * 8, 128), jnp.float32),
    ],
)


@functools.partial(
    pl.kernel,
    out_type=jax.ShapeDtypeStruct((BATCH,), jnp.float32),
    mesh=_mesh,
    compiler_params=pltpu.CompilerParams(
        needs_layout_passes=False, use_tc_tiling_on_sc=False
    ),
    scratch_types=dict(
        uidx_v=pltpu.VMEM((NCHUNK, CHUNK), jnp.int32),
        iidx_v=pltpu.VMEM((NCHUNK, CHUNK), jnp.int32),
        uidx_k=pltpu.VMEM((K, NCHUNK, CHUNK), jnp.int32),
        iidx_k=pltpu.VMEM((K, NCHUNK, CHUNK), jnp.int32),
        uidx_c=pltpu.VMEM((NCHUNK, CHUNK), jnp.int32),
        iidx_c=pltpu.VMEM((NCHUNK, CHUNK), jnp.int32),
        u_cols=pltpu.VMEM((K, BPW), jnp.float32),
        v_cols=pltpu.VMEM((K, BPW), jnp.float32),
        bu_rows=pltpu.VMEM((BPW,), jnp.float32),
        bi_rows=pltpu.VMEM((BPW,), jnp.float32),
        bu_tail=pltpu.VMEM((BTAIL,), jnp.float32),
        bi_tail=pltpu.VMEM((BTAIL,), jnp.float32),
        bias_v=pltpu.VMEM((L,), jnp.float32),
        out_v=pltpu.VMEM((BPW,), jnp.float32),
        sem=pltpu.SemaphoreType.DMA,
    ),
)
def _mf_sc(
    uidx_hbm,
    iidx_hbm,
    u_flat,
    v_flat,
    bu_main_hbm,
    bi_main_hbm,
    bu_tail_hbm,
    bi_tail_hbm,
    bias16,
    out_hbm,
    *,
    uidx_v,
    iidx_v,
    uidx_k,
    iidx_k,
    uidx_c,
    iidx_c,
    u_cols,
    v_cols,
    bu_rows,
    bi_rows,
    bu_tail,
    bi_tail,
    bias_v,
    out_v,
    sem,
):
    wid = lax.axis_index("s") * NC + lax.axis_index("c")

    pltpu.sync_copy(uidx_hbm.at[wid], uidx_v)
    pltpu.sync_copy(iidx_hbm.at[wid], iidx_v)
    pltpu.sync_copy(bu_tail_hbm, bu_tail)
    pltpu.sync_copy(bi_tail_hbm, bi_tail)
    pltpu.sync_copy(bias16, bias_v)

    bmax = jnp.full((L,), BMAIN - 1, jnp.int32)
    for c in range(NCHUNK):
        for b in range(SUB):
            s = pl.ds(b * L, L)
            base = uidx_v[c, s]
            uidx_c[c, s] = jnp.minimum(base, bmax)
            rb = base + (base >> 7) * 896
            for k in range(K):
                uidx_k[k, c, s] = rb + ((k // 8) * HALF + (k % 8) * 128)
            base = iidx_v[c, s]
            iidx_c[c, s] = jnp.minimum(base, bmax)
            rb = base + (base >> 7) * 896
            for k in range(K):
                iidx_k[k, c, s] = rb + ((k // 8) * HALF + (k % 8) * 128)

    descs = []
    for c in range(NCHUNK):
        rows = pl.ds(c * CHUNK, CHUNK)
        descs.append(pltpu.async_copy(bu_main_hbm.at[uidx_c.at[c]], bu_rows.at[rows], sem))
        descs.append(pltpu.async_copy(bi_main_hbm.at[iidx_c.at[c]], bi_rows.at[rows], sem))
        for k in range(K):
            descs.append(
                pltpu.async_copy(u_flat.at[uidx_k.at[k, c]], u_cols.at[k, rows], sem)
            )
            descs.append(
                pltpu.async_copy(v_flat.at[iidx_k.at[k, c]], v_cols.at[k, rows], sem)
            )
    for d in descs:
        d.wait()

    bias_vec = bias_v[...]
    bcut = jnp.full((L,), BMAIN, jnp.int32)
    tmax = jnp.full((L,), BTAIL - 1, jnp.int32)
    zero = jnp.zeros((L,), jnp.int32)

    def blk(i, carry):
        b = pl.ds(i * L, L)
        c = i // SUB
        s = pl.ds((i % SUB) * L, L)
        uix = uidx_v[c, s]
        iix = iidx_v[c, s]
        bu = bu_rows[b]
        bi = bi_rows[b]
        ut = plsc.load_gather(bu_tail, [jnp.clip(uix - bcut, zero, tmax)])
        it = plsc.load_gather(bi_tail, [jnp.clip(iix - bcut, zero, tmax)])
        bu = jnp.where(uix >= bcut, ut, bu)
        bi = jnp.where(iix >= bcut, it, bi)
        acc = bias_vec + bu + bi
        for k in range(K):
            acc = acc + u_cols[k, b] * v_cols[k, b]
        out_v[b] = acc
        return carry

    lax.fori_loop(0, NBLK, blk, 0)

    pltpu.sync_copy(out_v, out_hbm.at[pl.ds(wid * BPW, BPW)])


def kernel(train_x, user_emb, item_emb, bias_user, bias_item, bias):
    uidx = train_x[:, 0].reshape(NW, NCHUNK, CHUNK).astype(jnp.int32)
    iidx = train_x[:, 1].reshape(NW, NCHUNK, CHUNK).astype(jnp.int32)
    bias16 = jnp.broadcast_to(bias.astype(jnp.float32), (L,))
    u_dt, v_dt = _detile(user_emb.T, item_emb.T)
    u_flat = u_dt.reshape(2 * HALF)
    v_flat = v_dt.reshape(2 * HALF)
    bu_main = bias_user[:BMAIN].reshape(BMAIN)
    bi_main = bias_item[:BMAIN].reshape(BMAIN)
    bu_tail = bias_user[BMAIN:].reshape(BTAIL)
    bi_tail = bias_item[BMAIN:].reshape(BTAIL)
    return _mf_sc(
        uidx, iidx, u_flat, v_flat, bu_main, bi_main, bu_tail, bi_tail, bias16
    )
